# baseline (speedup 1.0000x reference)
import functools

import jax
import jax.numpy as jnp
from jax import lax
from jax.experimental import pallas as pl
from jax.experimental.pallas import tpu as pltpu
from jax.experimental.pallas import tpu_sc as plsc

N_NODES = 100000
N_PAIRS = 65536
FEAT = 320
HIDDEN = 128

NUM_WORKERS = 32
ROWS_PER_WORKER = N_PAIRS // NUM_WORKERS
CHUNK = 128
NUM_CHUNKS = ROWS_PER_WORKER // CHUNK


def _proj_block(d_ref, p_ref, w1a_ref, w1b_ref, b1_ref, a_ref, b_ref):
    a_ref[...] = jnp.dot(d_ref[...], w1a_ref[...],
                         preferred_element_type=jnp.float32) + b1_ref[...]
    b_ref[...] = jnp.dot(p_ref[...], w1b_ref[...],
                         preferred_element_type=jnp.float32)


def _tc_project(d, p, W1a, W1b, b1):
    block = 2000
    grid = (N_NODES // block,)
    return pl.pallas_call(
        _proj_block,
        grid=grid,
        in_specs=[
            pl.BlockSpec((block, FEAT), lambda i: (i, 0)),
            pl.BlockSpec((block, FEAT), lambda i: (i, 0)),
            pl.BlockSpec((FEAT, HIDDEN), lambda i: (0, 0)),
            pl.BlockSpec((FEAT, HIDDEN), lambda i: (0, 0)),
            pl.BlockSpec((1, HIDDEN), lambda i: (0, 0)),
        ],
        out_specs=[
            pl.BlockSpec((block, HIDDEN), lambda i: (i, 0)),
            pl.BlockSpec((block, HIDDEN), lambda i: (i, 0)),
        ],
        out_shape=[
            jax.ShapeDtypeStruct((N_NODES, HIDDEN), jnp.float32),
            jax.ShapeDtypeStruct((N_NODES, HIDDEN), jnp.float32),
        ],
    )(d, p, W1a, W1b, b1)


def _sc_gather_add(di, pi, a, b):
    mesh = plsc.VectorSubcoreMesh(core_axis_name="c", subcore_axis_name="s")

    @functools.partial(
        pl.kernel,
        mesh=mesh,
        out_type=jax.ShapeDtypeStruct((N_PAIRS, HIDDEN), jnp.float32),
        scratch_types=[
            pltpu.VMEM((CHUNK,), jnp.int32),
            pltpu.VMEM((CHUNK,), jnp.int32),
            pltpu.VMEM((CHUNK, HIDDEN), jnp.float32),
            pltpu.SemaphoreType.DMA,
        ],
    )
    def gather_kernel(di_hbm, pi_hbm, a_hbm, b_hbm, hpre_hbm,
                      idx_d, idx_p, rows_v, sem):
        wid = lax.axis_index("s") * 2 + lax.axis_index("c")
        base = wid * ROWS_PER_WORKER

        def body(i, carry):
            off = base + i * CHUNK
            pltpu.sync_copy(di_hbm.at[pl.ds(off, CHUNK)], idx_d)
            pltpu.sync_copy(pi_hbm.at[pl.ds(off, CHUNK)], idx_p)
            pltpu.async_copy(a_hbm.at[idx_d], rows_v, sem).wait()
            pltpu.async_copy(b_hbm.at[idx_p], rows_v, sem, add=True).wait()
            pltpu.sync_copy(rows_v, hpre_hbm.at[pl.ds(off, CHUNK)])
            return carry

        lax.fori_loop(0, NUM_CHUNKS, body, None)

    return gather_kernel(di, pi, a, b)


def _head_block(h_ref, w2_ref, b2_ref, out_ref):
    h = jnp.tanh(h_ref[...])
    logits = jnp.dot(h, w2_ref[...], preferred_element_type=jnp.float32)
    logits += b2_ref[...]
    m = jnp.max(logits, axis=1, keepdims=True)
    z = logits - m
    lse = jnp.log(jnp.sum(jnp.exp(z), axis=1, keepdims=True))
    out_ref[...] = z - lse


def _tc_head(hpre, W2, b2):
    block = 4096
    grid = (N_PAIRS // block,)
    return pl.pallas_call(
        _head_block,
        grid=grid,
        in_specs=[
            pl.BlockSpec((block, HIDDEN), lambda i: (i, 0)),
            pl.BlockSpec((HIDDEN, 2), lambda i: (0, 0)),
            pl.BlockSpec((1, 2), lambda i: (0, 0)),
        ],
        out_specs=pl.BlockSpec((block, 2), lambda i: (i, 0)),
        out_shape=jax.ShapeDtypeStruct((N_PAIRS, 2), jnp.float32),
    )(hpre, W2, b2)


def kernel(graph, dataset_index, iftrain, d, p, W1, b1, W2, b2):
    del graph, iftrain
    di = dataset_index[:, 0].astype(jnp.int32)
    pi = dataset_index[:, 1].astype(jnp.int32)
    a, b = _tc_project(d, p, W1[:FEAT], W1[FEAT:], b1.reshape(1, HIDDEN))
    hpre = _sc_gather_add(di, pi, a, b)
    return _tc_head(hpre, W2, b2.reshape(1, 2))


# speedup vs baseline: 1.0602x; 1.0602x over previous
import functools

import jax
import jax.numpy as jnp
from jax import lax
from jax.experimental import pallas as pl
from jax.experimental.pallas import tpu as pltpu
from jax.experimental.pallas import tpu_sc as plsc

N_NODES = 100000
N_PAIRS = 65536
FEAT = 320
HIDDEN = 128

NUM_WORKERS = 32
ROWS_PER_WORKER = N_PAIRS // NUM_WORKERS
CHUNK = 128
NUM_CHUNKS = ROWS_PER_WORKER // CHUNK


def _proj_block(d_ref, p_ref, w1a_ref, w1b_ref, b1_ref, a_ref, b_ref):
    a_ref[...] = jnp.dot(d_ref[...], w1a_ref[...],
                         preferred_element_type=jnp.float32) + b1_ref[...]
    b_ref[...] = jnp.dot(p_ref[...], w1b_ref[...],
                         preferred_element_type=jnp.float32)


def _tc_project(d, p, W1a, W1b, b1):
    block = 4000
    grid = (N_NODES // block,)
    return pl.pallas_call(
        _proj_block,
        grid=grid,
        in_specs=[
            pl.BlockSpec((block, FEAT), lambda i: (i, 0)),
            pl.BlockSpec((block, FEAT), lambda i: (i, 0)),
            pl.BlockSpec((FEAT, HIDDEN), lambda i: (0, 0)),
            pl.BlockSpec((FEAT, HIDDEN), lambda i: (0, 0)),
            pl.BlockSpec((1, HIDDEN), lambda i: (0, 0)),
        ],
        out_specs=[
            pl.BlockSpec((block, HIDDEN), lambda i: (i, 0)),
            pl.BlockSpec((block, HIDDEN), lambda i: (i, 0)),
        ],
        out_shape=[
            jax.ShapeDtypeStruct((N_NODES, HIDDEN), jnp.float32),
            jax.ShapeDtypeStruct((N_NODES, HIDDEN), jnp.float32),
        ],
    )(d, p, W1a, W1b, b1)


def _sc_gather_add(di, pi, a, b):
    mesh = plsc.VectorSubcoreMesh(core_axis_name="c", subcore_axis_name="s")

    @functools.partial(
        pl.kernel,
        mesh=mesh,
        out_type=jax.ShapeDtypeStruct((N_PAIRS, HIDDEN), jnp.float32),
        scratch_types=[
            pltpu.VMEM((ROWS_PER_WORKER,), jnp.int32),
            pltpu.VMEM((ROWS_PER_WORKER,), jnp.int32),
            pltpu.VMEM((CHUNK, HIDDEN), jnp.float32),
            pltpu.VMEM((CHUNK, HIDDEN), jnp.float32),
            pltpu.SemaphoreType.DMA,
            pltpu.SemaphoreType.DMA,
        ],
    )
    def gather_kernel(di_hbm, pi_hbm, a_hbm, b_hbm, hpre_hbm,
                      idx_d, idx_p, buf0, buf1, sem0, sem1):
        wid = lax.axis_index("s") * 2 + lax.axis_index("c")
        base = wid * ROWS_PER_WORKER
        pltpu.sync_copy(di_hbm.at[pl.ds(base, ROWS_PER_WORKER)], idx_d)
        pltpu.sync_copy(pi_hbm.at[pl.ds(base, ROWS_PER_WORKER)], idx_p)

        def body(i, carry):
            o0 = 2 * i * CHUNK
            o1 = o0 + CHUNK
            c0 = pltpu.async_copy(a_hbm.at[idx_d.at[pl.ds(o0, CHUNK)]],
                                  buf0, sem0)
            c1 = pltpu.async_copy(a_hbm.at[idx_d.at[pl.ds(o1, CHUNK)]],
                                  buf1, sem1)
            c0.wait()
            a0 = pltpu.async_copy(b_hbm.at[idx_p.at[pl.ds(o0, CHUNK)]],
                                  buf0, sem0, add=True)
            c1.wait()
            a1 = pltpu.async_copy(b_hbm.at[idx_p.at[pl.ds(o1, CHUNK)]],
                                  buf1, sem1, add=True)
            a0.wait()
            pltpu.sync_copy(buf0, hpre_hbm.at[pl.ds(base + o0, CHUNK)])
            a1.wait()
            pltpu.sync_copy(buf1, hpre_hbm.at[pl.ds(base + o1, CHUNK)])
            return carry

        lax.fori_loop(0, NUM_CHUNKS // 2, body, None)

    return gather_kernel(di, pi, a, b)


def _head_block(h_ref, w2_ref, b2_ref, out_ref):
    h = jnp.tanh(h_ref[...])
    logits = jnp.dot(h, w2_ref[...], preferred_element_type=jnp.float32)
    logits += b2_ref[...]
    m = jnp.max(logits, axis=1, keepdims=True)
    z = logits - m
    lse = jnp.log(jnp.sum(jnp.exp(z), axis=1, keepdims=True))
    out_ref[...] = z - lse


def _tc_head(hpre, W2, b2):
    block = 4096
    grid = (N_PAIRS // block,)
    return pl.pallas_call(
        _head_block,
        grid=grid,
        in_specs=[
            pl.BlockSpec((block, HIDDEN), lambda i: (i, 0)),
            pl.BlockSpec((HIDDEN, 2), lambda i: (0, 0)),
            pl.BlockSpec((1, 2), lambda i: (0, 0)),
        ],
        out_specs=pl.BlockSpec((block, 2), lambda i: (i, 0)),
        out_shape=jax.ShapeDtypeStruct((N_PAIRS, 2), jnp.float32),
    )(hpre, W2, b2)


def kernel(graph, dataset_index, iftrain, d, p, W1, b1, W2, b2):
    del graph, iftrain
    di = dataset_index[:, 0].astype(jnp.int32)
    pi = dataset_index[:, 1].astype(jnp.int32)
    a, b = _tc_project(d, p, W1[:FEAT], W1[FEAT:], b1.reshape(1, HIDDEN))
    hpre = _sc_gather_add(di, pi, a, b)
    return _tc_head(hpre, W2, b2.reshape(1, 2))
